# baseline (speedup 1.0000x reference)
import functools

import jax
import jax.numpy as jnp
from jax import lax
from jax.experimental import pallas as pl
from jax.experimental.pallas import tpu as pltpu
from jax.experimental.pallas import tpu_sc as plsc

B = 1024
L = 50
VOCAB = 100000
EMBED = 128
LANES = 16
ECHUNKS = EMBED // LANES


def _pool_sc(x_in, table):
    nc, ns = 2, 16
    nw = nc * ns
    b_per_w = B // nw

    mesh = plsc.VectorSubcoreMesh(core_axis_name="c", subcore_axis_name="s")

    @functools.partial(
        pl.kernel,
        mesh=mesh,
        out_type=jax.ShapeDtypeStruct((B, EMBED), jnp.float32),
        scratch_types=[
            pltpu.VMEM((b_per_w, L), jnp.int32),
            pltpu.VMEM((L, EMBED), jnp.float32),
            pltpu.VMEM((b_per_w, EMBED), jnp.float32),
            pltpu.SemaphoreType.DMA,
        ],
    )
    def pool_kernel(x_hbm, table_hbm, out_hbm, idx_v, rows_v, out_v, sem):
        wid = lax.axis_index("s") * nc + lax.axis_index("c")
        base = wid * b_per_w
        pltpu.sync_copy(x_hbm.at[pl.ds(base, b_per_w)], idx_v)

        def row_body(r, carry):
            pltpu.async_copy(table_hbm.at[idx_v.at[r]], rows_v, sem).wait()

            def l_body(l, accs):
                return tuple(
                    accs[e] + rows_v[l, pl.ds(e * LANES, LANES)]
                    for e in range(ECHUNKS)
                )

            zeros = tuple(jnp.zeros((LANES,), jnp.float32) for _ in range(ECHUNKS))
            accs = lax.fori_loop(0, L, l_body, zeros)
            for e in range(ECHUNKS):
                out_v[r, pl.ds(e * LANES, LANES)] = accs[e]
            return carry

        lax.fori_loop(0, b_per_w, row_body, 0)
        pltpu.sync_copy(out_v, out_hbm.at[pl.ds(base, b_per_w)])

    return pool_kernel(x_in, table)


def _fc_tc(x_sum, W, b2d):
    nblk = 1024
    grid = (pl.cdiv(VOCAB, nblk),)

    def mm_kernel(x_ref, w_ref, b_ref, o_ref):
        x = x_ref[...].astype(jnp.bfloat16)
        w = w_ref[...].astype(jnp.bfloat16)
        acc = lax.dot_general(
            x, w,
            dimension_numbers=(((1,), (1,)), ((), ())),
            preferred_element_type=jnp.float32,
        )
        o_ref[...] = acc + b_ref[...]

    return pl.pallas_call(
        mm_kernel,
        grid=grid,
        in_specs=[
            pl.BlockSpec((B, EMBED), lambda j: (0, 0)),
            pl.BlockSpec((nblk, EMBED), lambda j: (j, 0)),
            pl.BlockSpec((1, nblk), lambda j: (0, j)),
        ],
        out_specs=pl.BlockSpec((B, nblk), lambda j: (0, j)),
        out_shape=jax.ShapeDtypeStruct((B, VOCAB), jnp.float32),
    )(x_sum, W, b2d)


def kernel(x_in, table, W, b):
    x_sum = _pool_sc(x_in.astype(jnp.int32), table)
    return _fc_tc(x_sum, W, b.reshape(1, VOCAB))


# speedup vs baseline: 1.0418x; 1.0418x over previous
import functools

import jax
import jax.numpy as jnp
from jax import lax
from jax.experimental import pallas as pl
from jax.experimental.pallas import tpu as pltpu
from jax.experimental.pallas import tpu_sc as plsc

B = 1024
L = 50
VOCAB = 100000
EMBED = 128
LANES = 16
ECHUNKS = EMBED // LANES


def _pool_sc(x_in, table):
    nc, ns = 2, 16
    nw = nc * ns
    b_per_w = B // nw

    mesh = plsc.VectorSubcoreMesh(core_axis_name="c", subcore_axis_name="s")

    @functools.partial(
        pl.kernel,
        mesh=mesh,
        out_type=jax.ShapeDtypeStruct((B, EMBED), jnp.float32),
        scratch_types=[
            pltpu.VMEM((b_per_w, L), jnp.int32),
            pltpu.VMEM((L, EMBED), jnp.float32),
            pltpu.VMEM((b_per_w, EMBED), jnp.float32),
            pltpu.SemaphoreType.DMA,
        ],
    )
    def pool_kernel(x_hbm, table_hbm, out_hbm, idx_v, rows_v, out_v, sem):
        wid = lax.axis_index("s") * nc + lax.axis_index("c")
        base = wid * b_per_w
        pltpu.sync_copy(x_hbm.at[pl.ds(base, b_per_w)], idx_v)

        def row_body(r, carry):
            pltpu.async_copy(table_hbm.at[idx_v.at[r]], rows_v, sem).wait()

            def l_body(l, accs):
                return tuple(
                    accs[e] + rows_v[l, pl.ds(e * LANES, LANES)]
                    for e in range(ECHUNKS)
                )

            zeros = tuple(jnp.zeros((LANES,), jnp.float32) for _ in range(ECHUNKS))
            accs = lax.fori_loop(0, L, l_body, zeros)
            for e in range(ECHUNKS):
                out_v[r, pl.ds(e * LANES, LANES)] = accs[e]
            return carry

        lax.fori_loop(0, b_per_w, row_body, 0)
        pltpu.sync_copy(out_v, out_hbm.at[pl.ds(base, b_per_w)])

    return pool_kernel(x_in, table)


def _fc_tc(x_sum, W, b2d):
    nblk = 4096
    grid = (pl.cdiv(VOCAB, nblk),)

    def mm_kernel(x_ref, w_ref, b_ref, o_ref):
        x = x_ref[...].astype(jnp.bfloat16)
        w = w_ref[...].astype(jnp.bfloat16)
        acc = lax.dot_general(
            x, w,
            dimension_numbers=(((1,), (1,)), ((), ())),
            preferred_element_type=jnp.float32,
        )
        o_ref[...] = acc + b_ref[...]

    return pl.pallas_call(
        mm_kernel,
        grid=grid,
        in_specs=[
            pl.BlockSpec((B, EMBED), lambda j: (0, 0)),
            pl.BlockSpec((nblk, EMBED), lambda j: (j, 0)),
            pl.BlockSpec((1, nblk), lambda j: (0, j)),
        ],
        out_specs=pl.BlockSpec((B, nblk), lambda j: (0, j)),
        out_shape=jax.ShapeDtypeStruct((B, VOCAB), jnp.float32),
    )(x_sum, W, b2d)


def kernel(x_in, table, W, b):
    x_sum = _pool_sc(x_in.astype(jnp.int32), table)
    return _fc_tc(x_sum, W, b.reshape(1, VOCAB))


# speedup vs baseline: 2.8458x; 2.7317x over previous
import functools

import jax
import jax.numpy as jnp
from jax import lax
from jax.experimental import pallas as pl
from jax.experimental.pallas import tpu as pltpu
from jax.experimental.pallas import tpu_sc as plsc

B = 1024
L = 50
VOCAB = 100000
EMBED = 128
LANES = 16
ECHUNKS = EMBED // LANES


def _pool_sc(x_in, table):
    nc, ns = 2, 16
    nw = nc * ns
    b_per_w = B // nw

    mesh = plsc.VectorSubcoreMesh(core_axis_name="c", subcore_axis_name="s")

    @functools.partial(
        pl.kernel,
        mesh=mesh,
        out_type=jax.ShapeDtypeStruct((B, EMBED), jnp.float32),
        scratch_types=[
            pltpu.VMEM((b_per_w, L), jnp.int32),
            pltpu.VMEM((L, EMBED), jnp.float32),
            pltpu.VMEM((b_per_w, EMBED), jnp.float32),
            pltpu.SemaphoreType.DMA,
        ],
    )
    def pool_kernel(x_hbm, table_hbm, out_hbm, idx_v, rows_v, out_v, sem):
        wid = lax.axis_index("s") * nc + lax.axis_index("c")
        base = wid * b_per_w
        pltpu.sync_copy(x_hbm.at[pl.ds(base, b_per_w)], idx_v)

        def row_body(r, carry):
            pltpu.async_copy(table_hbm.at[idx_v.at[r]], rows_v, sem).wait()

            def l_body(l, accs):
                return tuple(
                    accs[e] + rows_v[l, pl.ds(e * LANES, LANES)]
                    for e in range(ECHUNKS)
                )

            zeros = tuple(jnp.zeros((LANES,), jnp.float32) for _ in range(ECHUNKS))
            accs = lax.fori_loop(0, L, l_body, zeros)
            for e in range(ECHUNKS):
                out_v[r, pl.ds(e * LANES, LANES)] = accs[e]
            return carry

        lax.fori_loop(0, b_per_w, row_body, 0)
        pltpu.sync_copy(out_v, out_hbm.at[pl.ds(base, b_per_w)])

    return pool_kernel(x_in, table)


def _fc_tc(x_sum, W, b2d):
    nblk = 2048
    grid = (pl.cdiv(VOCAB, nblk),)

    def mm_kernel(w_ref, x_ref, b_ref, o_ref):
        w = w_ref[...].astype(jnp.bfloat16)
        x = x_ref[...].astype(jnp.bfloat16)
        acc = lax.dot_general(
            w, x,
            dimension_numbers=(((1,), (1,)), ((), ())),
            preferred_element_type=jnp.float32,
        )
        bb = b_ref[...].astype(jnp.bfloat16)
        ones = jnp.ones((1, B), jnp.bfloat16)
        acc = acc + lax.dot_general(
            bb, ones,
            dimension_numbers=(((0,), (0,)), ((), ())),
            preferred_element_type=jnp.float32,
        )
        o_ref[...] = acc

    yT = pl.pallas_call(
        mm_kernel,
        grid=grid,
        in_specs=[
            pl.BlockSpec((nblk, EMBED), lambda j: (j, 0)),
            pl.BlockSpec((B, EMBED), lambda j: (0, 0)),
            pl.BlockSpec((1, nblk), lambda j: (0, j)),
        ],
        out_specs=pl.BlockSpec((nblk, B), lambda j: (j, 0)),
        out_shape=jax.ShapeDtypeStruct((VOCAB, B), jnp.float32),
    )(W, x_sum, b2d)
    return yT.T


def kernel(x_in, table, W, b):
    x_sum = _pool_sc(x_in.astype(jnp.int32), table)
    return _fc_tc(x_sum, W, b.reshape(1, VOCAB))


# speedup vs baseline: 3.1850x; 1.1192x over previous
import functools

import jax
import jax.numpy as jnp
from jax import lax
from jax.experimental import pallas as pl
from jax.experimental.pallas import tpu as pltpu
from jax.experimental.pallas import tpu_sc as plsc

B = 1024
L = 50
VOCAB = 100000
EMBED = 128
LANES = 16
ECHUNKS = EMBED // LANES


def _pool_sc(x_flat, table):
    nc, ns = 2, 16
    nw = nc * ns
    b_per_w = B // nw
    n_chunks = b_per_w // 2
    cw = 2 * L

    mesh = plsc.VectorSubcoreMesh(core_axis_name="c", subcore_axis_name="s")

    @functools.partial(
        pl.kernel,
        mesh=mesh,
        out_type=jax.ShapeDtypeStruct((B, EMBED), jnp.float32),
        scratch_types=[
            pltpu.VMEM((n_chunks, cw), jnp.int32),
            pltpu.VMEM((cw, EMBED), jnp.float32),
            pltpu.VMEM((cw, EMBED), jnp.float32),
            pltpu.VMEM((b_per_w, EMBED), jnp.float32),
            pltpu.SemaphoreType.DMA,
            pltpu.SemaphoreType.DMA,
        ],
    )
    def pool_kernel(x_hbm, table_hbm, out_hbm, idx_v, buf0, buf1, out_v,
                    sem0, sem1):
        wid = lax.axis_index("s") * nc + lax.axis_index("c")
        base = wid * b_per_w
        pltpu.sync_copy(x_hbm.at[pl.ds(wid * n_chunks, n_chunks)], idx_v)

        def start(c, buf, sem):
            pltpu.async_copy(table_hbm.at[idx_v.at[c]], buf, sem)

        def wait(buf, sem):
            pltpu.make_async_copy(table_hbm.at[idx_v.at[0]], buf, sem).wait()

        def consume(buf, r0):
            def l_body(l, accs):
                lo = tuple(
                    accs[e] + buf[l, pl.ds(e * LANES, LANES)]
                    for e in range(ECHUNKS)
                )
                hi = tuple(
                    accs[ECHUNKS + e] + buf[L + l, pl.ds(e * LANES, LANES)]
                    for e in range(ECHUNKS)
                )
                return lo + hi

            zeros = tuple(
                jnp.zeros((LANES,), jnp.float32) for _ in range(2 * ECHUNKS))
            accs = lax.fori_loop(0, L, l_body, zeros)
            for e in range(ECHUNKS):
                out_v[r0, pl.ds(e * LANES, LANES)] = accs[e]
                out_v[r0 + 1, pl.ds(e * LANES, LANES)] = accs[ECHUNKS + e]

        start(0, buf0, sem0)

        def pair_body(k, carry):
            c0 = 2 * k
            start(c0 + 1, buf1, sem1)
            wait(buf0, sem0)
            consume(buf0, 2 * c0)

            @pl.when(k < n_chunks // 2 - 1)
            def _():
                start(c0 + 2, buf0, sem0)

            wait(buf1, sem1)
            consume(buf1, 2 * (c0 + 1))
            return carry

        lax.fori_loop(0, n_chunks // 2, pair_body, 0)
        pltpu.sync_copy(out_v, out_hbm.at[pl.ds(base, b_per_w)])

    return pool_kernel(x_flat, table)


def _fc_tc(x_sum, W, b2d):
    nblk = 2048
    grid = (pl.cdiv(VOCAB, nblk),)

    def mm_kernel(w_ref, x_ref, b_ref, o_ref):
        w = w_ref[...].astype(jnp.bfloat16)
        x = x_ref[...].astype(jnp.bfloat16)
        acc = lax.dot_general(
            w, x,
            dimension_numbers=(((1,), (1,)), ((), ())),
            preferred_element_type=jnp.float32,
        )
        bb = b_ref[...].astype(jnp.bfloat16)
        ones = jnp.ones((1, B), jnp.bfloat16)
        acc = acc + lax.dot_general(
            bb, ones,
            dimension_numbers=(((0,), (0,)), ((), ())),
            preferred_element_type=jnp.float32,
        )
        o_ref[...] = acc

    yT = pl.pallas_call(
        mm_kernel,
        grid=grid,
        in_specs=[
            pl.BlockSpec((nblk, EMBED), lambda j: (j, 0)),
            pl.BlockSpec((B, EMBED), lambda j: (0, 0)),
            pl.BlockSpec((1, nblk), lambda j: (0, j)),
        ],
        out_specs=pl.BlockSpec((nblk, B), lambda j: (j, 0)),
        out_shape=jax.ShapeDtypeStruct((VOCAB, B), jnp.float32),
    )(W, x_sum, b2d)
    return yT.T


def kernel(x_in, table, W, b):
    x2 = x_in.astype(jnp.int32).reshape(B // 2, 2 * L)
    x_sum = _pool_sc(x2, table)
    return _fc_tc(x_sum, W, b.reshape(1, VOCAB))


# speedup vs baseline: 3.2323x; 1.0149x over previous
import functools

import jax
import jax.numpy as jnp
from jax import lax
from jax.experimental import pallas as pl
from jax.experimental.pallas import tpu as pltpu
from jax.experimental.pallas import tpu_sc as plsc

B = 1024
L = 50
VOCAB = 100000
EMBED = 128
LANES = 16
ECHUNKS = EMBED // LANES


def _pool_sc(x_flat, table):
    nc, ns = 2, 16
    nw = nc * ns
    b_per_w = B // nw
    n_chunks = b_per_w // 2
    cw = 2 * L

    mesh = plsc.VectorSubcoreMesh(core_axis_name="c", subcore_axis_name="s")

    @functools.partial(
        pl.kernel,
        mesh=mesh,
        out_type=jax.ShapeDtypeStruct((B, EMBED), jnp.float32),
        scratch_types=[
            pltpu.VMEM((n_chunks, cw), jnp.int32),
            pltpu.VMEM((cw, EMBED), jnp.float32),
            pltpu.VMEM((cw, EMBED), jnp.float32),
            pltpu.VMEM((b_per_w, EMBED), jnp.float32),
            pltpu.SemaphoreType.DMA,
            pltpu.SemaphoreType.DMA,
        ],
    )
    def pool_kernel(x_hbm, table_hbm, out_hbm, idx_v, buf0, buf1, out_v,
                    sem0, sem1):
        wid = lax.axis_index("s") * nc + lax.axis_index("c")
        base = wid * b_per_w
        pltpu.sync_copy(x_hbm.at[pl.ds(wid * n_chunks, n_chunks)], idx_v)

        def start(c, buf, sem):
            pltpu.async_copy(table_hbm.at[idx_v.at[c]], buf, sem)

        def wait(buf, sem):
            pltpu.make_async_copy(table_hbm.at[idx_v.at[0]], buf, sem).wait()

        def consume(buf, r0):
            def l_body(l, accs):
                lo = tuple(
                    accs[e] + buf[l, pl.ds(e * LANES, LANES)]
                    for e in range(ECHUNKS)
                )
                hi = tuple(
                    accs[ECHUNKS + e] + buf[L + l, pl.ds(e * LANES, LANES)]
                    for e in range(ECHUNKS)
                )
                return lo + hi

            zeros = tuple(
                jnp.zeros((LANES,), jnp.float32) for _ in range(2 * ECHUNKS))
            accs = lax.fori_loop(0, L, l_body, zeros)
            for e in range(ECHUNKS):
                out_v[r0, pl.ds(e * LANES, LANES)] = accs[e]
                out_v[r0 + 1, pl.ds(e * LANES, LANES)] = accs[ECHUNKS + e]

        start(0, buf0, sem0)

        def pair_body(k, carry):
            c0 = 2 * k
            start(c0 + 1, buf1, sem1)
            wait(buf0, sem0)
            consume(buf0, 2 * c0)

            @pl.when(k < n_chunks // 2 - 1)
            def _():
                start(c0 + 2, buf0, sem0)

            wait(buf1, sem1)
            consume(buf1, 2 * (c0 + 1))
            return carry

        lax.fori_loop(0, n_chunks // 2, pair_body, 0)
        pltpu.sync_copy(out_v, out_hbm.at[pl.ds(base, b_per_w)])

    return pool_kernel(x_flat, table)


def _fc_tc(x_sum, W, b2d):
    nblk = 4096
    grid = (pl.cdiv(VOCAB, nblk),)

    def mm_kernel(w_ref, x_ref, b_ref, o_ref):
        w = w_ref[...].astype(jnp.bfloat16)
        x = x_ref[...].astype(jnp.bfloat16)
        acc = lax.dot_general(
            w, x,
            dimension_numbers=(((1,), (1,)), ((), ())),
            preferred_element_type=jnp.float32,
        )
        bb = b_ref[...].astype(jnp.bfloat16)
        ones = jnp.ones((1, B), jnp.bfloat16)
        acc = acc + lax.dot_general(
            bb, ones,
            dimension_numbers=(((0,), (0,)), ((), ())),
            preferred_element_type=jnp.float32,
        )
        o_ref[...] = acc

    yT = pl.pallas_call(
        mm_kernel,
        grid=grid,
        in_specs=[
            pl.BlockSpec((nblk, EMBED), lambda j: (j, 0)),
            pl.BlockSpec((B, EMBED), lambda j: (0, 0)),
            pl.BlockSpec((1, nblk), lambda j: (0, j)),
        ],
        out_specs=pl.BlockSpec((nblk, B), lambda j: (j, 0)),
        out_shape=jax.ShapeDtypeStruct((VOCAB, B), jnp.float32),
    )(W, x_sum, b2d)
    return yT.T


def kernel(x_in, table, W, b):
    x2 = x_in.astype(jnp.int32).reshape(B // 2, 2 * L)
    x_sum = _pool_sc(x2, table)
    return _fc_tc(x_sum, W, b.reshape(1, VOCAB))


# speedup vs baseline: 3.2776x; 1.0140x over previous
import functools

import jax
import jax.numpy as jnp
from jax import lax
from jax.experimental import pallas as pl
from jax.experimental.pallas import tpu as pltpu
from jax.experimental.pallas import tpu_sc as plsc

B = 1024
L = 50
VOCAB = 100000
EMBED = 128
LANES = 16
ECHUNKS = EMBED // LANES


def _pool_sc(x_flat, table):
    nc, ns = 2, 16
    nw = nc * ns
    b_per_w = B // nw
    n_chunks = b_per_w // 2
    cw = 2 * L

    mesh = plsc.VectorSubcoreMesh(core_axis_name="c", subcore_axis_name="s")

    nbuf = 4
    scratch = [pltpu.VMEM((n_chunks, cw), jnp.int32)]
    scratch += [pltpu.VMEM((cw, EMBED), jnp.float32) for _ in range(nbuf)]
    scratch += [pltpu.VMEM((b_per_w, EMBED), jnp.float32)]
    scratch += [pltpu.SemaphoreType.DMA for _ in range(nbuf)]

    @functools.partial(
        pl.kernel,
        mesh=mesh,
        out_type=jax.ShapeDtypeStruct((B, EMBED), jnp.float32),
        scratch_types=scratch,
    )
    def pool_kernel(x_hbm, table_hbm, out_hbm, idx_v, *rest):
        bufs = rest[:nbuf]
        out_v = rest[nbuf]
        sems = rest[nbuf + 1:]
        wid = lax.axis_index("s") * nc + lax.axis_index("c")
        base = wid * b_per_w
        pltpu.sync_copy(x_hbm.at[pl.ds(wid * n_chunks, n_chunks)], idx_v)

        def start(c, j):
            pltpu.async_copy(table_hbm.at[idx_v.at[c]], bufs[j], sems[j])

        def wait(j):
            pltpu.make_async_copy(
                table_hbm.at[idx_v.at[0]], bufs[j], sems[j]).wait()

        def consume(buf, r0):
            def l_body(l, accs):
                lo = tuple(
                    accs[e] + buf[l, pl.ds(e * LANES, LANES)]
                    for e in range(ECHUNKS)
                )
                hi = tuple(
                    accs[ECHUNKS + e] + buf[L + l, pl.ds(e * LANES, LANES)]
                    for e in range(ECHUNKS)
                )
                return lo + hi

            zeros = tuple(
                jnp.zeros((LANES,), jnp.float32) for _ in range(2 * ECHUNKS))
            accs = lax.fori_loop(0, L, l_body, zeros)
            for e in range(ECHUNKS):
                out_v[r0, pl.ds(e * LANES, LANES)] = accs[e]
                out_v[r0 + 1, pl.ds(e * LANES, LANES)] = accs[ECHUNKS + e]

        for j in range(nbuf - 1):
            start(j, j)

        def ring_body(k, carry):
            for j in range(nbuf):
                c = nbuf * k + j
                wait(j)
                consume(bufs[j], 2 * c)

                @pl.when(c + (nbuf - 1) < n_chunks)
                def _():
                    start(c + (nbuf - 1), (j + nbuf - 1) % nbuf)

            return carry

        lax.fori_loop(0, n_chunks // nbuf, ring_body, 0)
        pltpu.sync_copy(out_v, out_hbm.at[pl.ds(base, b_per_w)])

    return pool_kernel(x_flat, table)


def _fc_tc(x_sum, W, b2d):
    nblk = 4096
    grid = (pl.cdiv(VOCAB, nblk),)

    def mm_kernel(w_ref, x_ref, b_ref, o_ref):
        w = w_ref[...].astype(jnp.bfloat16)
        x = x_ref[...].astype(jnp.bfloat16)
        acc = lax.dot_general(
            w, x,
            dimension_numbers=(((1,), (1,)), ((), ())),
            preferred_element_type=jnp.float32,
        )
        bb = b_ref[...].astype(jnp.bfloat16)
        ones = jnp.ones((1, B), jnp.bfloat16)
        acc = acc + lax.dot_general(
            bb, ones,
            dimension_numbers=(((0,), (0,)), ((), ())),
            preferred_element_type=jnp.float32,
        )
        o_ref[...] = acc

    yT = pl.pallas_call(
        mm_kernel,
        grid=grid,
        in_specs=[
            pl.BlockSpec((nblk, EMBED), lambda j: (j, 0)),
            pl.BlockSpec((B, EMBED), lambda j: (0, 0)),
            pl.BlockSpec((1, nblk), lambda j: (0, j)),
        ],
        out_specs=pl.BlockSpec((nblk, B), lambda j: (j, 0)),
        out_shape=jax.ShapeDtypeStruct((VOCAB, B), jnp.float32),
    )(W, x_sum, b2d)
    return yT.T


def kernel(x_in, table, W, b):
    x2 = x_in.astype(jnp.int32).reshape(B // 2, 2 * L)
    x_sum = _pool_sc(x2, table)
    return _fc_tc(x_sum, W, b.reshape(1, VOCAB))


# speedup vs baseline: 3.2905x; 1.0039x over previous
import functools

import jax
import jax.numpy as jnp
from jax import lax
from jax.experimental import pallas as pl
from jax.experimental.pallas import tpu as pltpu
from jax.experimental.pallas import tpu_sc as plsc

B = 1024
L = 50
VOCAB = 100000
EMBED = 128
LANES = 16
ECHUNKS = EMBED // LANES


def _pool_sc(x_flat, table):
    nc, ns = 2, 16
    nw = nc * ns
    b_per_w = B // nw
    n_chunks = b_per_w // 2
    cw = 2 * L

    mesh = plsc.VectorSubcoreMesh(core_axis_name="c", subcore_axis_name="s")

    nbuf = 8
    scratch = [pltpu.VMEM((n_chunks, cw), jnp.int32)]
    scratch += [pltpu.VMEM((cw, EMBED), jnp.float32) for _ in range(nbuf)]
    scratch += [pltpu.VMEM((b_per_w, EMBED), jnp.float32)]
    scratch += [pltpu.SemaphoreType.DMA for _ in range(nbuf)]

    @functools.partial(
        pl.kernel,
        mesh=mesh,
        out_type=jax.ShapeDtypeStruct((B, EMBED), jnp.float32),
        scratch_types=scratch,
    )
    def pool_kernel(x_hbm, table_hbm, out_hbm, idx_v, *rest):
        bufs = rest[:nbuf]
        out_v = rest[nbuf]
        sems = rest[nbuf + 1:]
        wid = lax.axis_index("s") * nc + lax.axis_index("c")
        base = wid * b_per_w
        pltpu.sync_copy(x_hbm.at[pl.ds(wid * n_chunks, n_chunks)], idx_v)

        def start(c, j):
            pltpu.async_copy(table_hbm.at[idx_v.at[c]], bufs[j], sems[j])

        def wait(j):
            pltpu.make_async_copy(
                table_hbm.at[idx_v.at[0]], bufs[j], sems[j]).wait()

        def consume(buf, r0):
            def l_body(l, accs):
                lo = tuple(
                    accs[e] + buf[l, pl.ds(e * LANES, LANES)]
                    for e in range(ECHUNKS)
                )
                hi = tuple(
                    accs[ECHUNKS + e] + buf[L + l, pl.ds(e * LANES, LANES)]
                    for e in range(ECHUNKS)
                )
                return lo + hi

            zeros = tuple(
                jnp.zeros((LANES,), jnp.float32) for _ in range(2 * ECHUNKS))
            accs = lax.fori_loop(0, L, l_body, zeros)
            for e in range(ECHUNKS):
                out_v[r0, pl.ds(e * LANES, LANES)] = accs[e]
                out_v[r0 + 1, pl.ds(e * LANES, LANES)] = accs[ECHUNKS + e]

        for j in range(nbuf - 1):
            start(j, j)

        def ring_body(k, carry):
            for j in range(nbuf):
                c = nbuf * k + j
                wait(j)
                consume(bufs[j], 2 * c)

                @pl.when(c + (nbuf - 1) < n_chunks)
                def _():
                    start(c + (nbuf - 1), (j + nbuf - 1) % nbuf)

            return carry

        lax.fori_loop(0, n_chunks // nbuf, ring_body, 0)
        pltpu.sync_copy(out_v, out_hbm.at[pl.ds(base, b_per_w)])

    return pool_kernel(x_flat, table)


def _fc_tc(x_sum, W, b2d):
    nblk = 4096
    grid = (pl.cdiv(VOCAB, nblk),)

    def mm_kernel(w_ref, x_ref, b_ref, o_ref):
        w = w_ref[...].astype(jnp.bfloat16)
        x = x_ref[...].astype(jnp.bfloat16)
        acc = lax.dot_general(
            w, x,
            dimension_numbers=(((1,), (1,)), ((), ())),
            preferred_element_type=jnp.float32,
        )
        bb = b_ref[...].astype(jnp.bfloat16)
        ones = jnp.ones((1, B), jnp.bfloat16)
        acc = acc + lax.dot_general(
            bb, ones,
            dimension_numbers=(((0,), (0,)), ((), ())),
            preferred_element_type=jnp.float32,
        )
        o_ref[...] = acc

    yT = pl.pallas_call(
        mm_kernel,
        grid=grid,
        in_specs=[
            pl.BlockSpec((nblk, EMBED), lambda j: (j, 0)),
            pl.BlockSpec((B, EMBED), lambda j: (0, 0)),
            pl.BlockSpec((1, nblk), lambda j: (0, j)),
        ],
        out_specs=pl.BlockSpec((nblk, B), lambda j: (j, 0)),
        out_shape=jax.ShapeDtypeStruct((VOCAB, B), jnp.float32),
    )(W, x_sum, b2d)
    return yT.T


def kernel(x_in, table, W, b):
    x2 = x_in.astype(jnp.int32).reshape(B // 2, 2 * L)
    x_sum = _pool_sc(x2, table)
    return _fc_tc(x_sum, W, b.reshape(1, VOCAB))


# speedup vs baseline: 3.2973x; 1.0021x over previous
import functools

import jax
import jax.numpy as jnp
from jax import lax
from jax.experimental import pallas as pl
from jax.experimental.pallas import tpu as pltpu
from jax.experimental.pallas import tpu_sc as plsc

B = 1024
L = 50
VOCAB = 100000
EMBED = 128
LANES = 16
ECHUNKS = EMBED // LANES


def _pool_sc(x_flat, table):
    nc, ns = 2, 16
    nw = nc * ns
    b_per_w = B // nw
    n_chunks = b_per_w // 2
    cw = 2 * L

    mesh = plsc.VectorSubcoreMesh(core_axis_name="c", subcore_axis_name="s")

    nbuf = 8
    scratch = [pltpu.VMEM((n_chunks, cw), jnp.int32)]
    scratch += [pltpu.VMEM((cw, EMBED), jnp.float32) for _ in range(nbuf)]
    scratch += [pltpu.VMEM((b_per_w, EMBED), jnp.float32)]
    scratch += [pltpu.SemaphoreType.DMA for _ in range(nbuf)]

    @functools.partial(
        pl.kernel,
        mesh=mesh,
        out_type=jax.ShapeDtypeStruct((B, EMBED), jnp.float32),
        scratch_types=scratch,
    )
    def pool_kernel(x_hbm, table_hbm, out_hbm, idx_v, *rest):
        bufs = rest[:nbuf]
        out_v = rest[nbuf]
        sems = rest[nbuf + 1:]
        wid = lax.axis_index("s") * nc + lax.axis_index("c")
        base = wid * b_per_w
        pltpu.sync_copy(x_hbm.at[pl.ds(wid * n_chunks, n_chunks)], idx_v)

        def start(c, j):
            pltpu.async_copy(table_hbm.at[idx_v.at[c]], bufs[j], sems[j])

        def wait(j):
            pltpu.make_async_copy(
                table_hbm.at[idx_v.at[0]], bufs[j], sems[j]).wait()

        def consume(buf, r0):
            def l_body(l, accs):
                lo = tuple(
                    accs[e] + buf[l, pl.ds(e * LANES, LANES)]
                    for e in range(ECHUNKS)
                )
                hi = tuple(
                    accs[ECHUNKS + e] + buf[L + l, pl.ds(e * LANES, LANES)]
                    for e in range(ECHUNKS)
                )
                return lo + hi

            zeros = tuple(
                jnp.zeros((LANES,), jnp.float32) for _ in range(2 * ECHUNKS))
            accs = lax.fori_loop(0, L, l_body, zeros)
            for e in range(ECHUNKS):
                out_v[r0, pl.ds(e * LANES, LANES)] = accs[e]
                out_v[r0 + 1, pl.ds(e * LANES, LANES)] = accs[ECHUNKS + e]

        for j in range(nbuf - 1):
            start(j, j)

        def ring_body(k, carry):
            for j in range(nbuf):
                c = nbuf * k + j
                wait(j)
                consume(bufs[j], 2 * c)

                @pl.when(c + (nbuf - 1) < n_chunks)
                def _():
                    start(c + (nbuf - 1), (j + nbuf - 1) % nbuf)

            return carry

        lax.fori_loop(0, n_chunks // nbuf, ring_body, 0)
        pltpu.sync_copy(out_v, out_hbm.at[pl.ds(base, b_per_w)])

    return pool_kernel(x_flat, table)


def _fc_tc(x_sum, W, b2d):
    nblk = 5120
    grid = (pl.cdiv(VOCAB, nblk),)

    def mm_kernel(w_ref, x_ref, b_ref, o_ref):
        w = w_ref[...].astype(jnp.bfloat16)
        x = x_ref[...].astype(jnp.bfloat16)
        acc = lax.dot_general(
            w, x,
            dimension_numbers=(((1,), (1,)), ((), ())),
            preferred_element_type=jnp.float32,
        )
        bb = b_ref[...].astype(jnp.bfloat16)
        ones = jnp.ones((1, B), jnp.bfloat16)
        acc = acc + lax.dot_general(
            bb, ones,
            dimension_numbers=(((0,), (0,)), ((), ())),
            preferred_element_type=jnp.float32,
        )
        o_ref[...] = acc

    yT = pl.pallas_call(
        mm_kernel,
        grid=grid,
        in_specs=[
            pl.BlockSpec((nblk, EMBED), lambda j: (j, 0)),
            pl.BlockSpec((B, EMBED), lambda j: (0, 0)),
            pl.BlockSpec((1, nblk), lambda j: (0, j)),
        ],
        out_specs=pl.BlockSpec((nblk, B), lambda j: (j, 0)),
        out_shape=jax.ShapeDtypeStruct((VOCAB, B), jnp.float32),
    )(W, x_sum, b2d)
    return yT.T


def kernel(x_in, table, W, b):
    x2 = x_in.astype(jnp.int32).reshape(B // 2, 2 * L)
    x_sum = _pool_sc(x2, table)
    return _fc_tc(x_sum, W, b.reshape(1, VOCAB))


# speedup vs baseline: 3.3021x; 1.0015x over previous
import functools

import jax
import jax.numpy as jnp
from jax import lax
from jax.experimental import pallas as pl
from jax.experimental.pallas import tpu as pltpu
from jax.experimental.pallas import tpu_sc as plsc

B = 1024
L = 50
VOCAB = 100000
EMBED = 128
LANES = 16
ECHUNKS = EMBED // LANES


def _pool_sc(x2, table):
    nc, ns = 2, 16
    nw = nc * ns
    b_per_w = B // nw
    n_chunks = b_per_w // 2
    cw = 2 * L

    mesh = plsc.VectorSubcoreMesh(core_axis_name="c", subcore_axis_name="s")

    nbuf = 8
    scratch = [pltpu.VMEM((n_chunks, cw), jnp.int32)]
    scratch += [pltpu.VMEM((cw, EMBED), jnp.float32) for _ in range(nbuf)]
    scratch += [pltpu.VMEM((b_per_w, EMBED), jnp.float32)]
    scratch += [pltpu.SemaphoreType.DMA for _ in range(nbuf)]

    @functools.partial(
        pl.kernel,
        mesh=mesh,
        out_type=jax.ShapeDtypeStruct((B, EMBED), jnp.float32),
        scratch_types=scratch,
    )
    def pool_kernel(x_hbm, table_hbm, out_hbm, idx_v, *rest):
        bufs = rest[:nbuf]
        out_v = rest[nbuf]
        sems = rest[nbuf + 1:]
        wid = lax.axis_index("s") * nc + lax.axis_index("c")
        base = wid * b_per_w
        pltpu.sync_copy(x_hbm.at[pl.ds(wid * n_chunks, n_chunks)], idx_v)

        def start(c, j):
            pltpu.async_copy(table_hbm.at[idx_v.at[c]], bufs[j], sems[j])

        def wait(j):
            pltpu.make_async_copy(
                table_hbm.at[idx_v.at[0]], bufs[j], sems[j]).wait()

        def consume(buf, r0):
            def l_body(l, accs):
                lo = tuple(
                    accs[e] + buf[l, pl.ds(e * LANES, LANES)]
                    for e in range(ECHUNKS)
                )
                hi = tuple(
                    accs[ECHUNKS + e] + buf[L + l, pl.ds(e * LANES, LANES)]
                    for e in range(ECHUNKS)
                )
                return lo + hi

            zeros = tuple(
                jnp.zeros((LANES,), jnp.float32) for _ in range(2 * ECHUNKS))
            accs = lax.fori_loop(0, L, l_body, zeros)
            for e in range(ECHUNKS):
                out_v[r0, pl.ds(e * LANES, LANES)] = accs[e]
                out_v[r0 + 1, pl.ds(e * LANES, LANES)] = accs[ECHUNKS + e]

        for j in range(nbuf - 1):
            start(j, j)

        def ring_body(k, carry):
            for j in range(nbuf):
                c = nbuf * k + j
                wait(j)
                consume(bufs[j], 2 * c)

                @pl.when(c + (nbuf - 1) < n_chunks)
                def _():
                    start(c + (nbuf - 1), (j + nbuf - 1) % nbuf)

            return carry

        lax.fori_loop(0, n_chunks // nbuf, ring_body, 0)
        pltpu.sync_copy(out_v, out_hbm.at[pl.ds(base, b_per_w)])

    return pool_kernel(x2, table)


def _fc_tc(x_sum, W, b2d):
    nblk = 5120
    grid = (pl.cdiv(VOCAB, nblk),)

    def mm_kernel(w_ref, x_ref, b_ref, o_ref):
        w = w_ref[...].astype(jnp.bfloat16)
        x = x_ref[...].astype(jnp.bfloat16)
        acc = lax.dot_general(
            w, x,
            dimension_numbers=(((1,), (1,)), ((), ())),
            preferred_element_type=jnp.float32,
        )
        bb = b_ref[...].astype(jnp.bfloat16)
        ones = jnp.ones((1, B), jnp.bfloat16)
        acc = acc + lax.dot_general(
            bb, ones,
            dimension_numbers=(((0,), (0,)), ((), ())),
            preferred_element_type=jnp.float32,
        )
        o_ref[...] = acc

    yT = pl.pallas_call(
        mm_kernel,
        grid=grid,
        in_specs=[
            pl.BlockSpec((nblk, EMBED), lambda j: (j, 0)),
            pl.BlockSpec((B, EMBED), lambda j: (0, 0)),
            pl.BlockSpec((1, nblk), lambda j: (0, j)),
        ],
        out_specs=pl.BlockSpec((nblk, B), lambda j: (j, 0)),
        out_shape=jax.ShapeDtypeStruct((VOCAB, B), jnp.float32),
    )(W, x_sum, b2d)
    return yT.T


def kernel(x_in, table, W, b):
    x2 = x_in.astype(jnp.int32).reshape(B // 2, 2 * L)
    x_sum = _pool_sc(x2, table)
    return _fc_tc(x_sum, W, b.reshape(1, VOCAB))
